# Initial kernel scaffold; baseline (speedup 1.0000x reference)
import jax
import jax.numpy as jnp
from jax.experimental import pallas as pl


def kernel(feat0, feat1, feat2, feat3):
    raise NotImplementedError("write your pallas kernel here")


# speedup vs baseline: 5.8519x; 5.8519x over previous
import numpy as np
import jax
import jax.numpy as jnp
from jax.experimental import pallas as pl
from jax.experimental.pallas import tpu as pltpu

_OUT = 64
_CB = 96
_TOTAL_C = 1440


def _interp_matrix(in_size: int, out_size: int = _OUT) -> np.ndarray:
    scale = in_size / out_size
    s = np.maximum((np.arange(out_size, dtype=np.float64) + 0.5) * scale - 0.5, 0.0)
    i0 = np.floor(s)
    w = (s - i0).astype(np.float32)
    lo = np.clip(i0.astype(np.int64), 0, in_size - 1)
    hi = np.clip(i0.astype(np.int64) + 1, 0, in_size - 1)
    m = np.zeros((out_size, in_size), dtype=np.float32)
    np.add.at(m, (np.arange(out_size), lo), 1.0 - w)
    np.add.at(m, (np.arange(out_size), hi), w)
    return m


def _body_l0(buf_ref, x_ref, rxt_ref, o_ref):
    s = x_ref[0, :, 0::2, :] + x_ref[0, :, 1::2, :]
    y = jnp.dot(s.reshape(_CB * _OUT, 128), rxt_ref[...],
                preferred_element_type=jnp.float32)
    o_ref[0] = y.reshape(_CB, _OUT, _OUT)


def _body_l1(buf_ref, x_ref, o_ref):
    o_ref[0] = x_ref[0]


def _body_l2(buf_ref, x_ref, rxt_ref, o_ref):
    x = x_ref[0]
    y = jnp.dot(x.reshape(_CB * 32, 32), rxt_ref[...],
                preferred_element_type=jnp.float32).reshape(_CB, 32, _OUT)
    dn = jnp.concatenate([y[:, :1], y[:, :-1]], axis=1)
    up = jnp.concatenate([y[:, 1:], y[:, -1:]], axis=1)
    o_ref[0:1, :, 0::2, :] = (0.25 * dn + 0.75 * y)[None]
    o_ref[0:1, :, 1::2, :] = (0.75 * y + 0.25 * up)[None]


def _body_l3(buf_ref, x_ref, rxt_ref, o_ref):
    x = x_ref[0]
    y = jnp.dot(x.reshape(_CB * 16, 16), rxt_ref[...],
                preferred_element_type=jnp.float32).reshape(_CB, 16, _OUT)
    dn = jnp.concatenate([y[:, :1], y[:, :-1]], axis=1)
    up = jnp.concatenate([y[:, 1:], y[:, -1:]], axis=1)
    o_ref[0:1, :, 0::4, :] = (0.375 * dn + 0.625 * y)[None]
    o_ref[0:1, :, 1::4, :] = (0.125 * dn + 0.875 * y)[None]
    o_ref[0:1, :, 2::4, :] = (0.875 * y + 0.125 * up)[None]
    o_ref[0:1, :, 3::4, :] = (0.625 * y + 0.375 * up)[None]


def _level_call(body, feat, ch_offset, buf, rxt=None, interpret=False):
    B, C, H, W = feat.shape
    grid = (B, C // _CB)
    off_blocks = ch_offset // _CB
    in_specs = [pl.BlockSpec(memory_space=pl.ANY),
                pl.BlockSpec((1, _CB, H, W), lambda b, c: (b, c, 0, 0))]
    operands = [buf, feat]
    if rxt is not None:
        in_specs.append(pl.BlockSpec((W, _OUT), lambda b, c: (0, 0)))
        operands.append(rxt)
    out_spec = pl.BlockSpec((1, _CB, _OUT, _OUT),
                            lambda b, c: (b, c + off_blocks, 0, 0))
    return pl.pallas_call(
        body,
        grid=grid,
        in_specs=in_specs,
        out_specs=out_spec,
        out_shape=jax.ShapeDtypeStruct((B, _TOTAL_C, _OUT, _OUT), jnp.float32),
        input_output_aliases={0: 0},
        interpret=interpret,
    )(*operands)


def _init_body(x_ref, o_ref):
    pass


def _first_call(body, feat, rxt, interpret=False):
    B, C, H, W = feat.shape
    grid = (B, C // _CB)
    return pl.pallas_call(
        lambda x_ref, rxt_ref, o_ref: body(None, x_ref, rxt_ref, o_ref),
        grid=grid,
        in_specs=[pl.BlockSpec((1, _CB, H, W), lambda b, c: (b, c, 0, 0)),
                  pl.BlockSpec((W, _OUT), lambda b, c: (0, 0))],
        out_specs=pl.BlockSpec((1, _CB, _OUT, _OUT), lambda b, c: (b, c, 0, 0)),
        out_shape=jax.ShapeDtypeStruct((B, _TOTAL_C, _OUT, _OUT), jnp.float32),
        interpret=interpret,
    )(feat, rxt)


_RXT0 = np.ascontiguousarray(0.5 * _interp_matrix(128).T, dtype=np.float32)
_RXT2 = np.ascontiguousarray(_interp_matrix(32).T, dtype=np.float32)
_RXT3 = np.ascontiguousarray(_interp_matrix(16).T, dtype=np.float32)


def kernel(feat0, feat1, feat2, feat3, interpret=False):
    buf = _first_call(_body_l0, feat0, _RXT0, interpret=interpret)
    buf = _level_call(_body_l1, feat1, 96, buf, interpret=interpret)
    buf = _level_call(_body_l2, feat2, 288, buf, rxt=_RXT2, interpret=interpret)
    buf = _level_call(_body_l3, feat3, 672, buf, rxt=_RXT3, interpret=interpret)
    return buf


# speedup vs baseline: 15.5862x; 2.6635x over previous
import numpy as np
import jax
import jax.numpy as jnp
from jax.experimental import pallas as pl
from jax.experimental.pallas import tpu as pltpu

_OUT = 64
_TOTAL_C = 1440
_B = 4


def _body(f0_ref, f1_ref, f2_ref, f3_ref, o_ref, scr_ref):
    h = pl.program_id(1)

    o_ref[0:1, :, :, 96:288] = f1_ref[0:1]

    t = (f0_ref[0:1, 0::2, 0::2, :] + f0_ref[0:1, 0::2, 1::2, :]
         + f0_ref[0:1, 1::2, 0::2, :] + f0_ref[0:1, 1::2, 1::2, :])
    o_ref[0:1, :, :, 0:96] = 0.25 * t

    def upsample(x_ref, scale, weights, ch_base):
        n, _, c = x_ref.shape
        half = scale // 2
        for k in range(c // 128):
            x = x_ref[:, :, 128 * k:128 * (k + 1)]
            xd = jnp.concatenate([x[:1], x[:-1]], axis=0)
            xu = jnp.concatenate([x[1:], x[-1:]], axis=0)
            rows = [w * xd + (1.0 - w) * x for w in weights[:half]]
            rows += [w * xu + (1.0 - w) * x for w in weights[half:]]
            for p, r in enumerate(rows):
                rd = jnp.concatenate([r[:, :1], r[:, :-1]], axis=1)
                ru = jnp.concatenate([r[:, 1:], r[:, -1:]], axis=1)
                cols = [w * rd + (1.0 - w) * r for w in weights[:half]]
                cols += [w * ru + (1.0 - w) * r for w in weights[half:]]
                for q, v in enumerate(cols):
                    if scale == 2:
                        scr_ref[p::2, q::2, :] = v
                    else:
                        scr_ref[p::4, q::4, :] = v
            lo = ch_base + 128 * k
            o_ref[0:1, :, :, lo:lo + 128] = scr_ref[pl.ds(32 * h, 32)][None]

    upsample(f2_ref[0], 2, (0.25, 0.25), 288)
    upsample(f3_ref[0], 4, (0.375, 0.125, 0.125, 0.375), 672)


def kernel(feat0, feat1, feat2, feat3):
    f0 = jnp.transpose(feat0, (0, 2, 3, 1))
    f1 = jnp.transpose(feat1, (0, 2, 3, 1))
    f2 = jnp.transpose(feat2, (0, 2, 3, 1))
    f3 = jnp.transpose(feat3, (0, 2, 3, 1))
    out = pl.pallas_call(
        _body,
        grid=(_B, 2),
        in_specs=[pl.BlockSpec((1, 64, 128, 96), lambda b, h: (b, h, 0, 0)),
                  pl.BlockSpec((1, 32, _OUT, 192), lambda b, h: (b, h, 0, 0)),
                  pl.BlockSpec((1, 32, 32, 384), lambda b, h: (b, 0, 0, 0)),
                  pl.BlockSpec((1, 16, 16, 768), lambda b, h: (b, 0, 0, 0))],
        out_specs=pl.BlockSpec((1, 32, _OUT, _TOTAL_C),
                               lambda b, h: (b, h, 0, 0)),
        out_shape=jax.ShapeDtypeStruct((_B, _OUT, _OUT, _TOTAL_C), jnp.float32),
        scratch_shapes=[pltpu.VMEM((_OUT, _OUT, 128), jnp.float32)],
    )(f0, f1, f2, f3)
    return jnp.transpose(out, (0, 3, 1, 2))


# speedup vs baseline: 24.8618x; 1.5951x over previous
import numpy as np
import jax
import jax.numpy as jnp
from jax.experimental import pallas as pl
from jax.experimental.pallas import tpu as pltpu

_OUT = 64
_TOTAL_C = 1440
_B = 4


def _pool_matrix() -> np.ndarray:
    m = np.zeros((128, _OUT), dtype=np.float32)
    idx = np.arange(_OUT)
    m[2 * idx, idx] = 0.25
    m[2 * idx + 1, idx] = 0.25
    return m


_RXT0 = _pool_matrix()


def _pool_body(x_ref, rxt_ref, o_ref):
    s = (x_ref[0:1, :, 0::2, :] + x_ref[0:1, :, 1::2, :])[0]
    y = jnp.dot(s.reshape(96 * _OUT, 128), rxt_ref[...],
                preferred_element_type=jnp.float32)
    o_ref[0] = y.reshape(96, _OUT, _OUT)


def _pool_feat0(feat0):
    return pl.pallas_call(
        _pool_body,
        grid=(_B,),
        in_specs=[pl.BlockSpec((1, 96, 128, 128), lambda b: (b, 0, 0, 0)),
                  pl.BlockSpec((128, _OUT), lambda b: (0, 0))],
        out_specs=pl.BlockSpec((1, 96, _OUT, _OUT), lambda b: (b, 0, 0, 0)),
        out_shape=jax.ShapeDtypeStruct((_B, 96, _OUT, _OUT), jnp.float32),
    )(feat0, _RXT0)


def _upsample_half(x, xd, xu, scale, weights, ch_base, o_ref, scr_ref):
    half = scale // 2
    c = x.shape[-1]
    for k in range(c // 128):
        lo = 128 * k
        xc, xdc, xuc = x[:, :, lo:lo + 128], xd[:, :, lo:lo + 128], xu[:, :, lo:lo + 128]
        rows = [w * xdc + (1.0 - w) * xc for w in weights[:half]]
        rows += [w * xuc + (1.0 - w) * xc for w in weights[half:]]
        for p, r in enumerate(rows):
            rd = jnp.concatenate([r[:, :1], r[:, :-1]], axis=1)
            ru = jnp.concatenate([r[:, 1:], r[:, -1:]], axis=1)
            cols = [w * rd + (1.0 - w) * r for w in weights[:half]]
            cols += [w * ru + (1.0 - w) * r for w in weights[half:]]
            for q, v in enumerate(cols):
                if scale == 2:
                    scr_ref[p::2, q::2, :] = v
                else:
                    scr_ref[p::4, q::4, :] = v
        o_ref[0:1, :, :, ch_base + lo:ch_base + lo + 128] = scr_ref[...][None]


def _body(p0_ref, f1_ref, f2_ref, f3_ref, o_ref, scr_ref):
    h = pl.program_id(1)

    o_ref[0:1, :, :, 0:96] = p0_ref[0:1]
    o_ref[0:1, :, :, 96:288] = f1_ref[0:1]

    def level(x_ref, scale, weights, ch_base):
        n = x_ref.shape[1]
        m = n // 2

        @pl.when(h == 0)
        def _():
            x = x_ref[0, 0:m]
            xd = jnp.concatenate([x[:1], x[:-1]], axis=0)
            xu = x_ref[0, 1:m + 1]
            _upsample_half(x, xd, xu, scale, weights, ch_base, o_ref, scr_ref)

        @pl.when(h == 1)
        def _():
            x = x_ref[0, m:n]
            xd = x_ref[0, m - 1:n - 1]
            xu = jnp.concatenate([x[1:], x[-1:]], axis=0)
            _upsample_half(x, xd, xu, scale, weights, ch_base, o_ref, scr_ref)

    level(f2_ref, 2, (0.25, 0.25), 288)
    level(f3_ref, 4, (0.375, 0.125, 0.125, 0.375), 672)


def kernel(feat0, feat1, feat2, feat3):
    p0 = jnp.transpose(_pool_feat0(feat0), (0, 2, 3, 1))
    f1 = jnp.transpose(feat1, (0, 2, 3, 1))
    f2 = jnp.transpose(feat2, (0, 2, 3, 1))
    f3 = jnp.transpose(feat3, (0, 2, 3, 1))
    out = pl.pallas_call(
        _body,
        grid=(_B, 2),
        in_specs=[pl.BlockSpec((1, 32, _OUT, 96), lambda b, h: (b, h, 0, 0)),
                  pl.BlockSpec((1, 32, _OUT, 192), lambda b, h: (b, h, 0, 0)),
                  pl.BlockSpec((1, 32, 32, 384), lambda b, h: (b, 0, 0, 0)),
                  pl.BlockSpec((1, 16, 16, 768), lambda b, h: (b, 0, 0, 0))],
        out_specs=pl.BlockSpec((1, 32, _OUT, _TOTAL_C),
                               lambda b, h: (b, h, 0, 0)),
        out_shape=jax.ShapeDtypeStruct((_B, _OUT, _OUT, _TOTAL_C), jnp.float32),
        scratch_shapes=[pltpu.VMEM((32, _OUT, 128), jnp.float32)],
    )(p0, f1, f2, f3)
    return jnp.transpose(out, (0, 3, 1, 2))
